# baseline (speedup 1.0000x reference)
import jax
import jax.numpy as jnp
from jax.experimental import pallas as pl
from jax.experimental.pallas import tpu as pltpu


def _bigru_kernel(x_ref, h0_ref, wih_ref, whh_ref, bih_ref, bhh_ref,
                  y_ref, hn_ref, gx_ref):
    T, B, E = x_ref.shape
    H = h0_ref.shape[-1]

    d = pl.program_id(0)
    base = d * (T - 1)
    rev = 1 - 2 * d

    wih = jnp.concatenate(
        [wih_ref[0], wih_ref[1], wih_ref[2]], axis=-1).astype(jnp.bfloat16)
    whh = jnp.concatenate(
        [whh_ref[0], whh_ref[1], whh_ref[2]], axis=-1).astype(jnp.bfloat16)
    bih = jnp.concatenate([bih_ref[0], bih_ref[1], bih_ref[2]], axis=-1)
    bhh = jnp.concatenate([bhh_ref[0], bhh_ref[1], bhh_ref[2]], axis=-1)

    x2d = x_ref[...].reshape(T * B, E)
    gx = jnp.dot(x2d, wih, preferred_element_type=jnp.float32)
    gx_ref[...] = (gx + bih).reshape(T, B, 3 * H)

    h = h0_ref[...]
    for t in range(T):
        s = base + rev * t
        gx_s = gx_ref[s]
        gh = jnp.dot(h.astype(jnp.bfloat16), whh,
                     preferred_element_type=jnp.float32) + bhh
        r = jax.nn.sigmoid(gx_s[:, :H] + gh[:, :H])
        z = jax.nn.sigmoid(gx_s[:, H:2 * H] + gh[:, H:2 * H])
        n = jnp.tanh(gx_s[:, 2 * H:] + r * gh[:, 2 * H:])
        h = (1.0 - z) * n + z * h
        y_ref[:, pl.ds(s, 1), :] = h[:, None, :]
    hn_ref[...] = h


def kernel(token_ids, h0, embedding, w_ih, w_hh, b_ih, b_hh):
    B, T = token_ids.shape
    E = embedding.shape[1]
    H = h0.shape[-1]

    x_tm = jnp.take(embedding, token_ids.T, axis=0).astype(jnp.bfloat16)

    output, hn = pl.pallas_call(
        _bigru_kernel,
        out_shape=(jax.ShapeDtypeStruct((B, T, 2 * H), jnp.float32),
                   jax.ShapeDtypeStruct((2, B, H), jnp.float32)),
        grid=(2,),
        in_specs=[
            pl.BlockSpec((T, B, E), lambda d: (0, 0, 0)),
            pl.BlockSpec((None, B, H), lambda d: (d, 0, 0)),
            pl.BlockSpec((None, 3, E, H), lambda d: (d, 0, 0, 0)),
            pl.BlockSpec((None, 3, H, H), lambda d: (d, 0, 0, 0)),
            pl.BlockSpec((None, 3, 1, H), lambda d: (d, 0, 0, 0)),
            pl.BlockSpec((None, 3, 1, H), lambda d: (d, 0, 0, 0)),
        ],
        out_specs=(pl.BlockSpec((B, T, H), lambda d: (0, 0, d)),
                   pl.BlockSpec((None, B, H), lambda d: (d, 0, 0))),
        scratch_shapes=[pltpu.VMEM((T, B, 3 * H), jnp.float32)],
        compiler_params=pltpu.CompilerParams(
            dimension_semantics=("parallel",)),
    )(x_tm, h0, w_ih, w_hh, b_ih, b_hh)

    return output, hn


# speedup vs baseline: 1.0041x; 1.0041x over previous
import jax
import jax.numpy as jnp
from jax.experimental import pallas as pl
from jax.experimental.pallas import tpu as pltpu

_NC = 4


def _bigru_kernel(x_ref, h0_ref, wih_ref, whh_ref, bih_ref, bhh_ref,
                  y_ref, hn_ref,
                  wih_s, whh_s, bx_s, bn_s, h_s, gx_ref):
    Tc, B, E = x_ref.shape
    H = h0_ref.shape[-1]

    d = pl.program_id(0)
    c = pl.program_id(1)
    nc = pl.num_programs(1)

    @pl.when(c == 0)
    def _init():
        wih_s[...] = jnp.concatenate(
            [wih_ref[0], wih_ref[1], wih_ref[2]], axis=-1).astype(jnp.bfloat16)
        whh_s[...] = jnp.concatenate(
            [whh_ref[0], whh_ref[1], whh_ref[2]], axis=-1).astype(jnp.bfloat16)
        bx_s[...] = jnp.concatenate(
            [bih_ref[0] + bhh_ref[0], bih_ref[1] + bhh_ref[1], bih_ref[2]],
            axis=-1)
        bn_s[...] = bhh_ref[2]
        h_s[...] = h0_ref[...]

    x2d = x_ref[...].reshape(Tc * B, E)
    gx = jnp.dot(x2d, wih_s[...], preferred_element_type=jnp.float32)
    gx_ref[...] = (gx + bx_s[...]).reshape(Tc, B, 3 * H)

    whh = whh_s[...]
    bn = bn_s[...]
    h = h_s[...]
    for t in range(Tc):
        s = t + d * (Tc - 1 - 2 * t)
        gx_s = gx_ref[s]
        gh = jnp.dot(h.astype(jnp.bfloat16), whh,
                     preferred_element_type=jnp.float32)
        r = jax.nn.sigmoid(gx_s[:, :H] + gh[:, :H])
        z = jax.nn.sigmoid(gx_s[:, H:2 * H] + gh[:, H:2 * H])
        n = jnp.tanh(gx_s[:, 2 * H:] + r * (gh[:, 2 * H:] + bn))
        h = n + z * (h - n)
        y_ref[:, pl.ds(s, 1), :] = h[:, None, :]
    h_s[...] = h

    @pl.when(c == nc - 1)
    def _fin():
        hn_ref[...] = h


def kernel(token_ids, h0, embedding, w_ih, w_hh, b_ih, b_hh):
    B, T = token_ids.shape
    E = embedding.shape[1]
    H = h0.shape[-1]
    Tc = T // _NC

    x_tm = jnp.take(embedding, token_ids.T, axis=0).astype(jnp.bfloat16)

    def xmap(d, c):
        return (c + d * (_NC - 1 - 2 * c), 0, 0)

    def ymap(d, c):
        return (0, c + d * (_NC - 1 - 2 * c), d)

    output, hn = pl.pallas_call(
        _bigru_kernel,
        out_shape=(jax.ShapeDtypeStruct((B, T, 2 * H), jnp.float32),
                   jax.ShapeDtypeStruct((2, B, H), jnp.float32)),
        grid=(2, _NC),
        in_specs=[
            pl.BlockSpec((Tc, B, E), xmap),
            pl.BlockSpec((None, B, H), lambda d, c: (d, 0, 0)),
            pl.BlockSpec((None, 3, E, H), lambda d, c: (d, 0, 0, 0)),
            pl.BlockSpec((None, 3, H, H), lambda d, c: (d, 0, 0, 0)),
            pl.BlockSpec((None, 3, 1, H), lambda d, c: (d, 0, 0, 0)),
            pl.BlockSpec((None, 3, 1, H), lambda d, c: (d, 0, 0, 0)),
        ],
        out_specs=(pl.BlockSpec((B, Tc, H), ymap),
                   pl.BlockSpec((None, B, H), lambda d, c: (d, 0, 0))),
        scratch_shapes=[
            pltpu.VMEM((E, 3 * H), jnp.bfloat16),
            pltpu.VMEM((H, 3 * H), jnp.bfloat16),
            pltpu.VMEM((1, 3 * H), jnp.float32),
            pltpu.VMEM((1, H), jnp.float32),
            pltpu.VMEM((B, H), jnp.float32),
            pltpu.VMEM((Tc, B, 3 * H), jnp.float32),
        ],
        compiler_params=pltpu.CompilerParams(
            dimension_semantics=("parallel", "arbitrary")),
    )(x_tm, h0, w_ih, w_hh, b_ih, b_hh)

    return output, hn


# speedup vs baseline: 1.0467x; 1.0425x over previous
import jax
import jax.numpy as jnp
from jax.experimental import pallas as pl
from jax.experimental.pallas import tpu as pltpu


def _bigru_kernel(x_ref, h0_ref, wih_ref, whh_ref, bih_ref, bhh_ref,
                  y_ref, hn_ref, gxf_ref, gxb_ref):
    T, Bh, E = x_ref.shape
    H = h0_ref.shape[-1]

    def fuse_w(ref, d):
        return jnp.concatenate(
            [ref[d, 0], ref[d, 1], ref[d, 2]], axis=-1).astype(jnp.bfloat16)

    def fuse_b(d):
        return jnp.concatenate(
            [bih_ref[d, 0] + bhh_ref[d, 0],
             bih_ref[d, 1] + bhh_ref[d, 1],
             bih_ref[d, 2]], axis=-1)

    whh_f = fuse_w(whh_ref, 0)
    whh_b = fuse_w(whh_ref, 1)
    bn_f = bhh_ref[0, 2]
    bn_b = bhh_ref[1, 2]

    x2d = x_ref[...].reshape(T * Bh, E)
    gxf_ref[...] = (
        jnp.dot(x2d, fuse_w(wih_ref, 0), preferred_element_type=jnp.float32)
        + fuse_b(0)).reshape(T, Bh, 3 * H)
    gxb_ref[...] = (
        jnp.dot(x2d, fuse_w(wih_ref, 1), preferred_element_type=jnp.float32)
        + fuse_b(1)).reshape(T, Bh, 3 * H)

    def step(h, gx_s, whh, bn):
        gh = jnp.dot(h.astype(jnp.bfloat16), whh,
                     preferred_element_type=jnp.float32)
        r = jax.nn.sigmoid(gx_s[:, :H] + gh[:, :H])
        z = jax.nn.sigmoid(gx_s[:, H:2 * H] + gh[:, H:2 * H])
        n = jnp.tanh(gx_s[:, 2 * H:] + r * (gh[:, 2 * H:] + bn))
        return n + z * (h - n)

    hf = h0_ref[0]
    hb = h0_ref[1]
    for t in range(T):
        sb = T - 1 - t
        hf = step(hf, gxf_ref[t], whh_f, bn_f)
        hb = step(hb, gxb_ref[sb], whh_b, bn_b)
        y_ref[:, t, :H] = hf
        y_ref[:, sb, H:] = hb
    hn_ref[0] = hf
    hn_ref[1] = hb


def kernel(token_ids, h0, embedding, w_ih, w_hh, b_ih, b_hh):
    B, T = token_ids.shape
    E = embedding.shape[1]
    H = h0.shape[-1]
    Bh = B // 2

    x_tm = jnp.take(embedding, token_ids.T, axis=0).astype(jnp.bfloat16)

    output, hn = pl.pallas_call(
        _bigru_kernel,
        out_shape=(jax.ShapeDtypeStruct((B, T, 2 * H), jnp.float32),
                   jax.ShapeDtypeStruct((2, B, H), jnp.float32)),
        grid=(2,),
        in_specs=[
            pl.BlockSpec((T, Bh, E), lambda p: (0, p, 0)),
            pl.BlockSpec((2, Bh, H), lambda p: (0, p, 0)),
            pl.BlockSpec((2, 3, E, H), lambda p: (0, 0, 0, 0)),
            pl.BlockSpec((2, 3, H, H), lambda p: (0, 0, 0, 0)),
            pl.BlockSpec((2, 3, 1, H), lambda p: (0, 0, 0, 0)),
            pl.BlockSpec((2, 3, 1, H), lambda p: (0, 0, 0, 0)),
        ],
        out_specs=(pl.BlockSpec((Bh, T, 2 * H), lambda p: (p, 0, 0)),
                   pl.BlockSpec((2, Bh, H), lambda p: (0, p, 0))),
        scratch_shapes=[
            pltpu.VMEM((T, Bh, 3 * H), jnp.float32),
            pltpu.VMEM((T, Bh, 3 * H), jnp.float32),
        ],
        compiler_params=pltpu.CompilerParams(
            dimension_semantics=("parallel",)),
    )(x_tm, h0, w_ih, w_hh, b_ih, b_hh)

    return output, hn


# speedup vs baseline: 1.1675x; 1.1154x over previous
import jax
import jax.numpy as jnp
from jax.experimental import pallas as pl
from jax.experimental.pallas import tpu as pltpu

_NCH = 8


def _bigru_kernel(tok_ref, emb_ref, h0_ref, wih_ref, whh_ref, bih_ref,
                  bhh_ref, y_ref, hn_ref, x_s, gxf_ref, gxb_ref, sem):
    T, Bh, E = x_s.shape
    H = h0_ref.shape[-1]
    B = tok_ref.shape[0] // T
    Tc = T // _NCH

    p = pl.program_id(0)
    row0 = p * Bh

    def issue_chunk(c):
        for t in range(c * Tc, (c + 1) * Tc):
            for b in range(Bh):
                idx = tok_ref[(row0 + b) * T + t]
                pltpu.make_async_copy(
                    emb_ref.at[idx], x_s.at[t, b], sem).start()

    def wait_chunk(c):
        for t in range(c * Tc, (c + 1) * Tc):
            for b in range(Bh):
                pltpu.make_async_copy(
                    x_s.at[t, b], x_s.at[t, b], sem).wait()

    def issue_pair(j):
        issue_chunk(j)
        issue_chunk(_NCH - 1 - j)

    def wait_pair(j):
        wait_chunk(j)
        wait_chunk(_NCH - 1 - j)

    issue_pair(0)

    def fuse_w(ref, d):
        return jnp.concatenate(
            [ref[d, 0], ref[d, 1], ref[d, 2]], axis=-1).astype(jnp.bfloat16)

    def fuse_b(d):
        return jnp.concatenate(
            [bih_ref[d, 0] + bhh_ref[d, 0],
             bih_ref[d, 1] + bhh_ref[d, 1],
             bih_ref[d, 2]], axis=-1)

    wih_f = fuse_w(wih_ref, 0)
    wih_b = fuse_w(wih_ref, 1)
    whh_f = fuse_w(whh_ref, 0)
    whh_b = fuse_w(whh_ref, 1)
    bx_f = fuse_b(0)
    bx_b = fuse_b(1)
    bn_f = bhh_ref[0, 2]
    bn_b = bhh_ref[1, 2]

    def gx_chunk(c):
        x2d = x_s[pl.ds(c * Tc, Tc)].reshape(Tc * Bh, E).astype(jnp.bfloat16)
        gxf_ref[pl.ds(c * Tc, Tc)] = (
            jnp.dot(x2d, wih_f, preferred_element_type=jnp.float32)
            + bx_f).reshape(Tc, Bh, 3 * H)
        gxb_ref[pl.ds(c * Tc, Tc)] = (
            jnp.dot(x2d, wih_b, preferred_element_type=jnp.float32)
            + bx_b).reshape(Tc, Bh, 3 * H)

    def step(h, gx_s, whh, bn):
        gh = jnp.dot(h.astype(jnp.bfloat16), whh,
                     preferred_element_type=jnp.float32)
        r = jax.nn.sigmoid(gx_s[:, :H] + gh[:, :H])
        z = jax.nn.sigmoid(gx_s[:, H:2 * H] + gh[:, H:2 * H])
        n = jnp.tanh(gx_s[:, 2 * H:] + r * (gh[:, 2 * H:] + bn))
        return n + z * (h - n)

    hf = h0_ref[0]
    hb = h0_ref[1]

    for k in range(_NCH):
        if k < _NCH // 2:
            wait_pair(k)
            if k + 1 < _NCH // 2:
                issue_pair(k + 1)
            gx_chunk(k)
            gx_chunk(_NCH - 1 - k)
        cb = _NCH - 1 - k
        for tt in range(Tc):
            t = k * Tc + tt
            sb = cb * Tc + (Tc - 1 - tt)
            hf = step(hf, gxf_ref[t], whh_f, bn_f)
            hb = step(hb, gxb_ref[sb], whh_b, bn_b)
            y_ref[:, t, :H] = hf
            y_ref[:, sb, H:] = hb
    hn_ref[0] = hf
    hn_ref[1] = hb


def kernel(token_ids, h0, embedding, w_ih, w_hh, b_ih, b_hh):
    B, T = token_ids.shape
    E = embedding.shape[1]
    H = h0.shape[-1]
    Bh = B // 2

    output, hn = pl.pallas_call(
        _bigru_kernel,
        out_shape=(jax.ShapeDtypeStruct((B, T, 2 * H), jnp.float32),
                   jax.ShapeDtypeStruct((2, B, H), jnp.float32)),
        grid_spec=pltpu.PrefetchScalarGridSpec(
            num_scalar_prefetch=1,
            grid=(2,),
            in_specs=[
                pl.BlockSpec(memory_space=pl.ANY),
                pl.BlockSpec((2, Bh, H), lambda p, tok: (0, p, 0)),
                pl.BlockSpec((2, 3, E, H), lambda p, tok: (0, 0, 0, 0)),
                pl.BlockSpec((2, 3, H, H), lambda p, tok: (0, 0, 0, 0)),
                pl.BlockSpec((2, 3, 1, H), lambda p, tok: (0, 0, 0, 0)),
                pl.BlockSpec((2, 3, 1, H), lambda p, tok: (0, 0, 0, 0)),
            ],
            out_specs=(pl.BlockSpec((Bh, T, 2 * H), lambda p, tok: (p, 0, 0)),
                       pl.BlockSpec((2, Bh, H), lambda p, tok: (0, p, 0))),
            scratch_shapes=[
                pltpu.VMEM((T, Bh, E), jnp.float32),
                pltpu.VMEM((T, Bh, 3 * H), jnp.float32),
                pltpu.VMEM((T, Bh, 3 * H), jnp.float32),
                pltpu.SemaphoreType.DMA,
            ],
        ),
        compiler_params=pltpu.CompilerParams(
            dimension_semantics=("parallel",),
            disable_bounds_checks=True),
    )(token_ids.reshape(B * T), embedding, h0, w_ih, w_hh, b_ih, b_hh)

    return output, hn


# speedup vs baseline: 1.1708x; 1.0029x over previous
import jax
import jax.numpy as jnp
from jax.experimental import pallas as pl
from jax.experimental.pallas import tpu as pltpu

_NCH = 8


def _bigru_kernel(tok_ref, emb_ref, h0_ref, wih_ref, whh_ref, bih_ref,
                  bhh_ref, y_ref, hn_ref, x_s, gxf_ref, gxb_ref, sem):
    T, Bh, E = x_s.shape
    H = h0_ref.shape[-1]
    B = tok_ref.shape[0] // T
    Tc = T // _NCH

    p = pl.program_id(0)
    row0 = p * Bh

    def issue_chunk(c):
        for t in range(c * Tc, (c + 1) * Tc):
            for b in range(Bh):
                idx = tok_ref[(row0 + b) * T + t]
                pltpu.make_async_copy(
                    emb_ref.at[idx], x_s.at[t, b], sem.at[c]).start()

    def wait_chunk(c):
        pltpu.make_async_copy(
            x_s.at[pl.ds(c * Tc, Tc)], x_s.at[pl.ds(c * Tc, Tc)],
            sem.at[c]).wait()

    def issue_pair(j):
        issue_chunk(j)
        issue_chunk(_NCH - 1 - j)

    def wait_pair(j):
        wait_chunk(j)
        wait_chunk(_NCH - 1 - j)

    issue_pair(0)

    def fuse_w(ref, d):
        return jnp.concatenate(
            [ref[d, 0], ref[d, 1], ref[d, 2]], axis=-1).astype(jnp.bfloat16)

    def fuse_b(d):
        return jnp.concatenate(
            [bih_ref[d, 0] + bhh_ref[d, 0],
             bih_ref[d, 1] + bhh_ref[d, 1],
             bih_ref[d, 2]], axis=-1)

    wih_f = fuse_w(wih_ref, 0)
    wih_b = fuse_w(wih_ref, 1)
    whh_f = fuse_w(whh_ref, 0)
    whh_b = fuse_w(whh_ref, 1)
    bx_f = fuse_b(0)
    bx_b = fuse_b(1)
    bn_f = bhh_ref[0, 2]
    bn_b = bhh_ref[1, 2]

    def gx_chunk(c):
        x2d = x_s[pl.ds(c * Tc, Tc)].reshape(Tc * Bh, E).astype(jnp.bfloat16)
        gxf_ref[pl.ds(c * Tc, Tc)] = (
            jnp.dot(x2d, wih_f, preferred_element_type=jnp.float32)
            + bx_f).reshape(Tc, Bh, 3 * H)
        gxb_ref[pl.ds(c * Tc, Tc)] = (
            jnp.dot(x2d, wih_b, preferred_element_type=jnp.float32)
            + bx_b).reshape(Tc, Bh, 3 * H)

    def step(h, gx_s, whh, bn):
        gh = jnp.dot(h.astype(jnp.bfloat16), whh,
                     preferred_element_type=jnp.float32)
        r = jax.nn.sigmoid(gx_s[:, :H] + gh[:, :H])
        z = jax.nn.sigmoid(gx_s[:, H:2 * H] + gh[:, H:2 * H])
        n = jnp.tanh(gx_s[:, 2 * H:] + r * (gh[:, 2 * H:] + bn))
        return n + z * (h - n)

    hf = h0_ref[0]
    hb = h0_ref[1]

    for k in range(_NCH):
        if k < _NCH // 2:
            wait_pair(k)
            if k + 1 < _NCH // 2:
                issue_pair(k + 1)
            gx_chunk(k)
            gx_chunk(_NCH - 1 - k)
        cb = _NCH - 1 - k
        for tt in range(Tc):
            t = k * Tc + tt
            sb = cb * Tc + (Tc - 1 - tt)
            hf = step(hf, gxf_ref[t], whh_f, bn_f)
            hb = step(hb, gxb_ref[sb], whh_b, bn_b)
            y_ref[:, t, :H] = hf
            y_ref[:, sb, H:] = hb
    hn_ref[0] = hf
    hn_ref[1] = hb


def kernel(token_ids, h0, embedding, w_ih, w_hh, b_ih, b_hh):
    B, T = token_ids.shape
    E = embedding.shape[1]
    H = h0.shape[-1]
    Bh = B // 2

    output, hn = pl.pallas_call(
        _bigru_kernel,
        out_shape=(jax.ShapeDtypeStruct((B, T, 2 * H), jnp.float32),
                   jax.ShapeDtypeStruct((2, B, H), jnp.float32)),
        grid_spec=pltpu.PrefetchScalarGridSpec(
            num_scalar_prefetch=1,
            grid=(2,),
            in_specs=[
                pl.BlockSpec(memory_space=pl.ANY),
                pl.BlockSpec((2, Bh, H), lambda p, tok: (0, p, 0)),
                pl.BlockSpec((2, 3, E, H), lambda p, tok: (0, 0, 0, 0)),
                pl.BlockSpec((2, 3, H, H), lambda p, tok: (0, 0, 0, 0)),
                pl.BlockSpec((2, 3, 1, H), lambda p, tok: (0, 0, 0, 0)),
                pl.BlockSpec((2, 3, 1, H), lambda p, tok: (0, 0, 0, 0)),
            ],
            out_specs=(pl.BlockSpec((Bh, T, 2 * H), lambda p, tok: (p, 0, 0)),
                       pl.BlockSpec((2, Bh, H), lambda p, tok: (0, p, 0))),
            scratch_shapes=[
                pltpu.VMEM((T, Bh, E), jnp.float32),
                pltpu.VMEM((T, Bh, 3 * H), jnp.float32),
                pltpu.VMEM((T, Bh, 3 * H), jnp.float32),
                pltpu.SemaphoreType.DMA((_NCH,)),
            ],
        ),
        compiler_params=pltpu.CompilerParams(
            dimension_semantics=("parallel",),
            disable_bounds_checks=True),
    )(token_ids.reshape(B * T), embedding, h0, w_ih, w_hh, b_ih, b_hh)

    return output, hn
